# baseline (speedup 1.0000x reference)
import functools

import jax
import jax.numpy as jnp
from jax import lax
from jax.experimental import pallas as pl
from jax.experimental.pallas import tpu as pltpu
from jax.experimental.pallas import tpu_sc as plsc

B = 4096
L = 50
LP = 56
D = 64
NUM_CLASSES = 128

NC = 2
NS = 16
NW = NC * NS
BPW = B // NW
TPW = BPW * LP
BPC = 2
NCHUNK = BPW // BPC
NBUF = 3

_F32_MAX = float(jnp.finfo(jnp.float32).max)

_mesh = plsc.VectorSubcoreMesh(
    core_axis_name="c", subcore_axis_name="s", num_cores=NC, num_subcores=NS
)


@functools.partial(
    pl.kernel,
    out_type=jax.ShapeDtypeStruct((B * D,), jnp.float32),
    mesh=_mesh,
    scratch_types=[
        pltpu.VMEM((TPW + 16,), jnp.int32),
        pltpu.VMEM((TPW,), jnp.int32),
        pltpu.VMEM((NBUF, BPC * LP, 128), jnp.float32),
        pltpu.VMEM((BPW,), jnp.int32),
        pltpu.VMEM((BPW, 128), jnp.float32),
        pltpu.VMEM((BPW * D,), jnp.float32),
        pltpu.SemaphoreType.DMA,
        pltpu.SemaphoreType.DMA,
        pltpu.SemaphoreType.DMA,
    ],
    compiler_params=pltpu.CompilerParams(use_tc_tiling_on_sc=True),
)
def _sc_pool(tokens_hbm, cat0_hbm, emb2_hbm, cat2_hbm, x_hbm,
             idx_v, sidx_v, rows_v, catidx_v, catrows_v, x_v, *sems):
    wid = lax.axis_index("s") * NC + lax.axis_index("c")

    pltpu.sync_copy(tokens_hbm.at[pl.ds(wid * TPW, TPW)], idx_v.at[pl.ds(0, TPW)])
    pltpu.sync_copy(cat0_hbm.at[pl.ds(wid * BPW, BPW)], catidx_v)
    pltpu.async_copy(cat2_hbm.at[catidx_v], catrows_v, sems[0]).wait()

    def prep_body(i, _):
        v = sidx_v[pl.ds(16 * i, 16)]
        sidx_v[pl.ds(16 * i, 16)] = lax.shift_right_logical(v, 1)
        return 0

    pltpu.sync_copy(tokens_hbm.at[pl.ds(wid * TPW, TPW)], sidx_v)
    lax.fori_loop(0, TPW // 16, prep_body, 0, unroll=8)

    lane = lax.iota(jnp.int32, 16)
    lo8 = lane < 8

    def compute_chunk(j, buf):
        for bb in range(BPC):
            b_loc = j * BPC + bb

            def pair_body(p, carry):
                a0, a1, a2, a3, cnt = carry
                tl = bb * LP + 2 * p
                ov = idx_v[pl.ds(j * BPC * LP + tl, 16)]
                oA = lax.shift_left(ov[0] & 1, 6)
                oB = lax.shift_left(ov[1] & 1, 6)
                rA0 = buf[tl, pl.ds(oA, 16)]
                rA1 = buf[tl, pl.ds(oA + 16, 16)]
                rA2 = buf[tl, pl.ds(oA + 32, 16)]
                rA3 = buf[tl, pl.ds(oA + 48, 16)]
                rB0 = buf[tl + 1, pl.ds(oB, 16)]
                rB1 = buf[tl + 1, pl.ds(oB + 16, 16)]
                rB2 = buf[tl + 1, pl.ds(oB + 32, 16)]
                rB3 = buf[tl + 1, pl.ds(oB + 48, 16)]
                sA = (rA0 + rA1) + (rA2 + rA3)
                sB = (rB0 + rB1) + (rB2 + rB3)
                sA = sA + sA[lane ^ 8]
                sB = sB + sB[lane ^ 8]
                u = jnp.where(lo8, sA, sB)
                for sh in (1, 2, 4):
                    u = u + u[lane ^ sh]
                cnt = cnt + jnp.where(u != 0.0, 1.0, 0.0)
                return (a0 + (rA0 + rB0), a1 + (rA1 + rB1),
                        a2 + (rA2 + rB2), a3 + (rA3 + rB3), cnt)

            z16 = jnp.zeros((16,), jnp.float32)
            a0, a1, a2, a3, cnt = lax.fori_loop(
                0, LP // 2, pair_body, (z16, z16, z16, z16, z16), unroll=7
            )
            cnt = cnt + cnt[lane ^ 8]
            rcp = 1.0 / cnt
            for k, a in enumerate((a0, a1, a2, a3)):
                y = a * rcp
                y = jnp.where(y != y, 0.0, y)
                y = jnp.minimum(jnp.maximum(y, -_F32_MAX), _F32_MAX)
                y = y + catrows_v[b_loc, pl.ds(16 * k, 16)]
                x_v[pl.ds(b_loc * D + 16 * k, 16)] = y

    def start(j, buf, sem):
        return pltpu.async_copy(
            emb2_hbm.at[sidx_v.at[pl.ds(j * BPC * LP, BPC * LP)]], buf, sem
        )

    for k in range(NBUF - 1):
        start(k, rows_v.at[k], sems[k])

    def ring_body(i, _):
        for k in range(NBUF):
            j = NBUF * i + k
            nj = j + NBUF - 1

            @pl.when(nj < NCHUNK)
            def _():
                start(nj, rows_v.at[(k + NBUF - 1) % NBUF],
                      sems[(k + NBUF - 1) % NBUF])

            @pl.when(j < NCHUNK)
            def _():
                pltpu.make_async_copy(
                    emb2_hbm.at[sidx_v.at[pl.ds(j * BPC * LP, BPC * LP)]],
                    rows_v.at[k], sems[k],
                ).wait()
                compute_chunk(j, rows_v.at[k])
        return 0

    lax.fori_loop(0, (NCHUNK + NBUF - 1) // NBUF, ring_body, 0)
    pltpu.sync_copy(x_v, x_hbm.at[pl.ds(wid * BPW * D, BPW * D)])


def _tc_matmul_body(x_ref, w_ref, b_ref, o_ref):
    o_ref[...] = (
        lax.dot_general(
            x_ref[...], w_ref[...], (((1,), (1,)), ((), ())),
            preferred_element_type=jnp.float32,
        )
        + b_ref[...]
    )


_tc_matmul = pl.pallas_call(
    _tc_matmul_body,
    out_shape=jax.ShapeDtypeStruct((B, NUM_CLASSES), jnp.float32),
)


def kernel(tokens, cat_0, emb_table, cat_table, fc_w, fc_b):
    tokens_p = jnp.pad(tokens.astype(jnp.int32), ((0, 0), (0, LP - L)))
    emb2 = emb_table.reshape(50000, 128)
    cat2 = jnp.pad(cat_table, ((0, 0), (0, 128 - D)))
    x = _sc_pool(tokens_p.reshape(-1), cat_0.astype(jnp.int32), emb2, cat2)
    return _tc_matmul(x.reshape(B, D), fc_w, fc_b[None, :])


# speedup vs baseline: 9.1888x; 9.1888x over previous
import functools

import jax
import jax.numpy as jnp
from jax import lax
from jax.experimental import pallas as pl
from jax.experimental.pallas import tpu as pltpu
from jax.experimental.pallas import tpu_sc as plsc

B = 4096
L = 50
D = 64
NUM_CLASSES = 128

NC = 2
NS = 16
NW = NC * NS
BPW = B // NW
BPC = 2
NCHUNK = BPW // BPC
NBUF = 4

_F32_MAX = float(jnp.finfo(jnp.float32).max)

_mesh = plsc.VectorSubcoreMesh(
    core_axis_name="c", subcore_axis_name="s", num_cores=NC, num_subcores=NS
)


@functools.partial(
    pl.kernel,
    out_type=jax.ShapeDtypeStruct((NW, BPW, D), jnp.float32),
    mesh=_mesh,
    scratch_types=[
        pltpu.VMEM((NCHUNK, BPC * L), jnp.int32),
        pltpu.VMEM((NBUF, BPC * L, D), jnp.float32),
        pltpu.VMEM((BPW,), jnp.int32),
        pltpu.VMEM((BPW, D), jnp.float32),
        pltpu.VMEM((BPW, D), jnp.float32),
        pltpu.SemaphoreType.DMA,
        pltpu.SemaphoreType.DMA,
        pltpu.SemaphoreType.DMA,
        pltpu.SemaphoreType.DMA,
    ],
    compiler_params=pltpu.CompilerParams(use_tc_tiling_on_sc=False),
)
def _sc_pool(tokens_hbm, cat0_hbm, emb_hbm, cat_hbm, x_hbm,
             idx_v, rows_v, catidx_v, catrows_v, x_v, *sems):
    wid = lax.axis_index("s") * NC + lax.axis_index("c")

    pltpu.sync_copy(tokens_hbm.at[wid], idx_v)
    pltpu.sync_copy(cat0_hbm.at[wid], catidx_v)
    pltpu.async_copy(cat_hbm.at[catidx_v], catrows_v, sems[0]).wait()

    lane = lax.iota(jnp.int32, 16)
    lo8 = lane < 8

    def compute_chunk(j, buf):
        for bb in range(BPC):
            b_loc = j * BPC + bb

            def pair_body(p, carry):
                a0, a1, a2, a3, cnt = carry
                tA = bb * L + 2 * p
                rA0 = buf[tA, pl.ds(0, 16)]
                rA1 = buf[tA, pl.ds(16, 16)]
                rA2 = buf[tA, pl.ds(32, 16)]
                rA3 = buf[tA, pl.ds(48, 16)]
                rB0 = buf[tA + 1, pl.ds(0, 16)]
                rB1 = buf[tA + 1, pl.ds(16, 16)]
                rB2 = buf[tA + 1, pl.ds(32, 16)]
                rB3 = buf[tA + 1, pl.ds(48, 16)]
                sA = (rA0 + rA1) + (rA2 + rA3)
                sB = (rB0 + rB1) + (rB2 + rB3)
                sA = sA + sA[lane ^ 8]
                sB = sB + sB[lane ^ 8]
                u = jnp.where(lo8, sA, sB)
                for sh in (1, 2, 4):
                    u = u + u[lane ^ sh]
                cnt = cnt + jnp.where(u != 0.0, 1.0, 0.0)
                return (a0 + (rA0 + rB0), a1 + (rA1 + rB1),
                        a2 + (rA2 + rB2), a3 + (rA3 + rB3), cnt)

            z16 = jnp.zeros((16,), jnp.float32)
            a0, a1, a2, a3, cnt = lax.fori_loop(
                0, L // 2, pair_body, (z16, z16, z16, z16, z16), unroll=5
            )
            cnt = cnt + cnt[lane ^ 8]
            rcp = 1.0 / cnt
            for k, a in enumerate((a0, a1, a2, a3)):
                y = a * rcp
                y = jnp.where(y != y, 0.0, y)
                y = jnp.minimum(jnp.maximum(y, -_F32_MAX), _F32_MAX)
                y = y + catrows_v[b_loc, pl.ds(16 * k, 16)]
                x_v[b_loc, pl.ds(16 * k, 16)] = y

    def start(j, buf, sem):
        return pltpu.async_copy(emb_hbm.at[idx_v.at[j]], buf, sem)

    for k in range(NBUF - 1):
        start(k, rows_v.at[k], sems[k])

    def ring_body(i, _):
        for k in range(NBUF):
            j = NBUF * i + k
            nj = j + NBUF - 1

            @pl.when(nj < NCHUNK)
            def _():
                start(nj, rows_v.at[(k + NBUF - 1) % NBUF],
                      sems[(k + NBUF - 1) % NBUF])

            pltpu.make_async_copy(
                emb_hbm.at[idx_v.at[j]], rows_v.at[k], sems[k]
            ).wait()
            compute_chunk(j, rows_v.at[k])
        return 0

    lax.fori_loop(0, NCHUNK // NBUF, ring_body, 0)
    pltpu.sync_copy(x_v, x_hbm.at[wid])


def _tc_matmul_body(x_ref, w_ref, b_ref, o_ref):
    o_ref[...] = (
        lax.dot_general(
            x_ref[...], w_ref[...], (((1,), (1,)), ((), ())),
            preferred_element_type=jnp.float32,
        )
        + b_ref[...]
    )


_tc_matmul = pl.pallas_call(
    _tc_matmul_body,
    out_shape=jax.ShapeDtypeStruct((B, NUM_CLASSES), jnp.float32),
)


def kernel(tokens, cat_0, emb_table, cat_table, fc_w, fc_b):
    tokens_r = tokens.reshape(NW, NCHUNK, BPC * L).astype(jnp.int32)
    cat_r = cat_0.reshape(NW, BPW).astype(jnp.int32)
    x = _sc_pool(tokens_r, cat_r, emb_table, cat_table)
    return _tc_matmul(x.reshape(B, D), fc_w, fc_b[None, :])
